# baseline (speedup 1.0000x reference)
import functools

import jax
import jax.numpy as jnp
from jax import lax
from jax.experimental import pallas as pl
from jax.experimental.pallas import tpu as pltpu
from jax.experimental.pallas import tpu_sc as plsc

_L = 16
_NC = 2
_NS = 16
_NW = _NC * _NS
_N = 8192 * 512
_PER_W = _N // _NW
_CHUNK = 16384
_NCHUNKS = _PER_W // _CHUNK


def _sc_table_gather(species_flat, conv):
    mesh = plsc.VectorSubcoreMesh(core_axis_name="c", subcore_axis_name="s")

    @functools.partial(
        pl.kernel,
        mesh=mesh,
        out_type=jax.ShapeDtypeStruct((_N,), jnp.int32),
        compiler_params=pltpu.CompilerParams(needs_layout_passes=False),
        scratch_types=[
            pltpu.VMEM((128,), jnp.int32),
            pltpu.VMEM((_CHUNK,), jnp.int32),
            pltpu.VMEM((_CHUNK,), jnp.int32),
            pltpu.VMEM((_CHUNK,), jnp.int32),
            pltpu.VMEM((_CHUNK,), jnp.int32),
            pltpu.SemaphoreType.DMA,
            pltpu.SemaphoreType.DMA,
            pltpu.SemaphoreType.DMA,
            pltpu.SemaphoreType.DMA,
        ],
    )
    def k(species_hbm, conv_hbm, out_hbm, conv_v,
          in0, in1, out0, out1, si0, si1, so0, so1):
        wid = lax.axis_index("s") * _NC + lax.axis_index("c")
        base = wid * _PER_W
        pltpu.sync_copy(conv_hbm, conv_v)

        in_bufs, out_bufs = (in0, in1), (out0, out1)
        in_sems, out_sems = (si0, si1), (so0, so1)

        def start_in(ci):
            return pltpu.async_copy(
                species_hbm.at[pl.ds(base + ci * _CHUNK, _CHUNK)],
                in_bufs[ci % 2], in_sems[ci % 2])

        def start_out(ci):
            return pltpu.async_copy(
                out_bufs[ci % 2],
                out_hbm.at[pl.ds(base + ci * _CHUNK, _CHUNK)],
                out_sems[ci % 2])

        def translate(src, dst):
            @plsc.parallel_loop(0, _CHUNK, _L, unroll=8)
            def body(i):
                idx = src[pl.ds(i, _L)]
                dst[pl.ds(i, _L)] = plsc.load_gather(conv_v, [idx])

        pending_in = {0: start_in(0)}
        pending_out = {}
        for ci in range(_NCHUNKS):
            b = ci % 2
            if ci + 1 < _NCHUNKS:
                pending_in[ci + 1] = start_in(ci + 1)
            pending_in.pop(ci).wait()
            if ci - 2 in pending_out:
                pending_out.pop(ci - 2).wait()
            translate(in_bufs[b], out_bufs[b])
            pending_out[ci] = start_out(ci)
        for ci in sorted(pending_out):
            pending_out.pop(ci).wait()

    return k(species_flat, conv)


def _tc_passthrough(coords):
    rows, feat = coords.shape[0], coords.shape[1] * coords.shape[2]
    flat = coords.reshape(rows, feat)
    block = rows // 16

    def body(x_ref, o_ref):
        o_ref[...] = x_ref[...]

    out = pl.pallas_call(
        body,
        grid=(16,),
        in_specs=[pl.BlockSpec((block, feat), lambda i: (i, 0))],
        out_specs=pl.BlockSpec((block, feat), lambda i: (i, 0)),
        out_shape=jax.ShapeDtypeStruct((rows, feat), coords.dtype),
    )(flat)
    return out.reshape(coords.shape)


def kernel(species, coordinates, conv_tensor):
    species_flat = species.astype(jnp.int32).reshape(-1)
    conv = jnp.pad(conv_tensor.astype(jnp.int32), (0, 128 - conv_tensor.shape[0]))
    out = _sc_table_gather(species_flat, conv)
    coords_out = _tc_passthrough(coordinates)
    return out.reshape(species.shape).astype(conv_tensor.dtype), coords_out


# speedup vs baseline: 5.6480x; 5.6480x over previous
import functools

import jax
import jax.numpy as jnp
from jax import lax
from jax.experimental import pallas as pl
from jax.experimental.pallas import tpu as pltpu
from jax.experimental.pallas import tpu_sc as plsc

_L = 16
_NC = 2
_NS = 16
_NW = _NC * _NS
_ROWS = 8192
_COLS = 512
_ROWS_PER_W = _ROWS // _NW
_CROWS = 32
_NCHUNKS = _ROWS_PER_W // _CROWS
_CELEMS = _CROWS * _COLS


def _sc_table_gather(species, conv):
    mesh = plsc.VectorSubcoreMesh(core_axis_name="c", subcore_axis_name="s")

    @functools.partial(
        pl.kernel,
        mesh=mesh,
        out_type=jax.ShapeDtypeStruct((_ROWS, _COLS), jnp.int32),
        compiler_params=pltpu.CompilerParams(needs_layout_passes=False),
        scratch_types=[
            pltpu.VMEM((128,), jnp.int32),
            pltpu.VMEM((_CROWS, _COLS), jnp.int32),
            pltpu.VMEM((_CROWS, _COLS), jnp.int32),
            pltpu.VMEM((_CROWS, _COLS), jnp.int32),
            pltpu.VMEM((_CROWS, _COLS), jnp.int32),
            pltpu.SemaphoreType.DMA,
            pltpu.SemaphoreType.DMA,
            pltpu.SemaphoreType.DMA,
            pltpu.SemaphoreType.DMA,
        ],
    )
    def k(species_hbm, conv_hbm, out_hbm, conv_v,
          in0, in1, out0, out1, si0, si1, so0, so1):
        wid = lax.axis_index("s") * _NC + lax.axis_index("c")
        base = wid * _ROWS_PER_W
        pltpu.sync_copy(conv_hbm, conv_v)

        in_bufs, out_bufs = (in0, in1), (out0, out1)
        in_sems, out_sems = (si0, si1), (so0, so1)

        def start_in(ci):
            return pltpu.async_copy(
                species_hbm.at[pl.ds(base + ci * _CROWS, _CROWS)],
                in_bufs[ci % 2], in_sems[ci % 2])

        def start_out(ci):
            return pltpu.async_copy(
                out_bufs[ci % 2],
                out_hbm.at[pl.ds(base + ci * _CROWS, _CROWS)],
                out_sems[ci % 2])

        def translate(src, dst):
            @plsc.parallel_loop(0, _CELEMS, _L, unroll=8)
            def body(i):
                r = i >> 9
                c = i & (_COLS - 1)
                idx = src[r, pl.ds(c, _L)]
                dst[r, pl.ds(c, _L)] = plsc.load_gather(conv_v, [idx])

        pending_in = {0: start_in(0)}
        pending_out = {}
        for ci in range(_NCHUNKS):
            b = ci % 2
            if ci + 1 < _NCHUNKS:
                pending_in[ci + 1] = start_in(ci + 1)
            pending_in.pop(ci).wait()
            if ci - 2 in pending_out:
                pending_out.pop(ci - 2).wait()
            translate(in_bufs[b], out_bufs[b])
            pending_out[ci] = start_out(ci)
        for ci in sorted(pending_out):
            pending_out.pop(ci).wait()

    return k(species, conv)


def kernel(species, coordinates, conv_tensor):
    conv = jnp.pad(conv_tensor.astype(jnp.int32), (0, 128 - conv_tensor.shape[0]))
    out = _sc_table_gather(species.astype(jnp.int32), conv)
    return out.astype(conv_tensor.dtype), coordinates


# speedup vs baseline: 5.6534x; 1.0010x over previous
import functools

import jax
import jax.numpy as jnp
from jax import lax
from jax.experimental import pallas as pl
from jax.experimental.pallas import tpu as pltpu
from jax.experimental.pallas import tpu_sc as plsc

_L = 16
_NC = 2
_NS = 16
_NW = _NC * _NS
_ROWS = 8192
_COLS = 512
_ROWS_PER_W = _ROWS // _NW
_CROWS = 32
_NCHUNKS = _ROWS_PER_W // _CROWS
_CELEMS = _CROWS * _COLS


def _sc_table_gather(species, conv):
    mesh = plsc.VectorSubcoreMesh(core_axis_name="c", subcore_axis_name="s")

    @functools.partial(
        pl.kernel,
        mesh=mesh,
        out_type=jax.ShapeDtypeStruct((_ROWS, _COLS), jnp.int32),
        compiler_params=pltpu.CompilerParams(needs_layout_passes=False),
        scratch_types=[
            pltpu.VMEM((128,), jnp.int32),
            pltpu.VMEM((_CROWS, _COLS), jnp.int32),
            pltpu.VMEM((_CROWS, _COLS), jnp.int32),
            pltpu.VMEM((_CROWS, _COLS), jnp.int32),
            pltpu.VMEM((_CROWS, _COLS), jnp.int32),
            pltpu.SemaphoreType.DMA,
            pltpu.SemaphoreType.DMA,
            pltpu.SemaphoreType.DMA,
            pltpu.SemaphoreType.DMA,
        ],
    )
    def k(species_hbm, conv_hbm, out_hbm, conv_v,
          in0, in1, out0, out1, si0, si1, so0, so1):
        wid = lax.axis_index("s") * _NC + lax.axis_index("c")
        base = wid * _ROWS_PER_W
        pltpu.sync_copy(conv_hbm, conv_v)

        in_bufs, out_bufs = (in0, in1), (out0, out1)
        in_sems, out_sems = (si0, si1), (so0, so1)

        def start_in(ci):
            return pltpu.async_copy(
                species_hbm.at[pl.ds(base + ci * _CROWS, _CROWS)],
                in_bufs[ci % 2], in_sems[ci % 2])

        def start_out(ci):
            return pltpu.async_copy(
                out_bufs[ci % 2],
                out_hbm.at[pl.ds(base + ci * _CROWS, _CROWS)],
                out_sems[ci % 2])

        def translate(src, dst):
            @plsc.parallel_loop(0, _CELEMS, _L, unroll=8)
            def body(i):
                r = i >> 9
                c = i & (_COLS - 1)
                idx = src[r, pl.ds(c, _L)]
                dst[r, pl.ds(c, _L)] = plsc.load_gather(conv_v, [idx])

        pending_in = {0: start_in(0)}
        pending_out = {}
        for ci in range(_NCHUNKS):
            b = ci % 2
            if ci + 1 < _NCHUNKS:
                pending_in[ci + 1] = start_in(ci + 1)
            pending_in.pop(ci).wait()
            if ci - 2 in pending_out:
                pending_out.pop(ci - 2).wait()
            translate(in_bufs[b], out_bufs[b])
            pending_out[ci] = start_out(ci)
        for ci in sorted(pending_out):
            pending_out.pop(ci).wait()

    return k(species, conv)


def kernel(species, coordinates, conv_tensor):
    conv = jnp.pad(conv_tensor.astype(jnp.int32), (0, 128 - conv_tensor.shape[0]))
    coords_out = jnp.copy(coordinates)
    out = _sc_table_gather(species.astype(jnp.int32), conv)
    return out.astype(conv_tensor.dtype), coords_out


# speedup vs baseline: 5.6951x; 1.0074x over previous
import functools

import jax
import jax.numpy as jnp
from jax import lax
from jax.experimental import pallas as pl
from jax.experimental.pallas import tpu as pltpu
from jax.experimental.pallas import tpu_sc as plsc

_L = 16
_NC = 2
_NS = 16
_NW = _NC * _NS
_ROWS = 8192
_COLS = 512
_ROWS_PER_W = _ROWS // _NW
_CROWS = 16
_NCHUNKS = _ROWS_PER_W // _CROWS
_CELEMS = _CROWS * _COLS
_NBUF = 4


def _sc_table_gather(species, conv):
    mesh = plsc.VectorSubcoreMesh(core_axis_name="c", subcore_axis_name="s")

    @functools.partial(
        pl.kernel,
        mesh=mesh,
        out_type=jax.ShapeDtypeStruct((_ROWS, _COLS), jnp.int32),
        compiler_params=pltpu.CompilerParams(needs_layout_passes=False),
        scratch_types=(
            [pltpu.VMEM((128,), jnp.int32)]
            + [pltpu.VMEM((_CROWS, _COLS), jnp.int32)] * (2 * _NBUF)
            + [pltpu.SemaphoreType.DMA] * (2 * _NBUF)
        ),
    )
    def k(species_hbm, conv_hbm, out_hbm, conv_v, *rest):
        in_bufs = rest[:_NBUF]
        out_bufs = rest[_NBUF:2 * _NBUF]
        in_sems = rest[2 * _NBUF:3 * _NBUF]
        out_sems = rest[3 * _NBUF:]
        wid = lax.axis_index("s") * _NC + lax.axis_index("c")
        base = wid * _ROWS_PER_W
        pltpu.sync_copy(conv_hbm, conv_v)

        def start_in(ci):
            return pltpu.async_copy(
                species_hbm.at[pl.ds(base + ci * _CROWS, _CROWS)],
                in_bufs[ci % _NBUF], in_sems[ci % _NBUF])

        def start_out(ci):
            return pltpu.async_copy(
                out_bufs[ci % _NBUF],
                out_hbm.at[pl.ds(base + ci * _CROWS, _CROWS)],
                out_sems[ci % _NBUF])

        def translate(src, dst):
            @plsc.parallel_loop(0, _CELEMS, _L, unroll=8)
            def body(i):
                r = i >> 9
                c = i & (_COLS - 1)
                idx = src[r, pl.ds(c, _L)]
                dst[r, pl.ds(c, _L)] = plsc.load_gather(conv_v, [idx])

        pending_in = {ci: start_in(ci) for ci in range(_NBUF - 1)}
        pending_out = {}
        for ci in range(_NCHUNKS):
            b = ci % _NBUF
            if ci + _NBUF - 1 < _NCHUNKS:
                pending_in[ci + _NBUF - 1] = start_in(ci + _NBUF - 1)
            pending_in.pop(ci).wait()
            if ci - _NBUF in pending_out:
                pending_out.pop(ci - _NBUF).wait()
            translate(in_bufs[b], out_bufs[b])
            pending_out[ci] = start_out(ci)
        for ci in sorted(pending_out):
            pending_out.pop(ci).wait()

    return k(species, conv)


def kernel(species, coordinates, conv_tensor):
    conv = jnp.pad(conv_tensor.astype(jnp.int32), (0, 128 - conv_tensor.shape[0]))
    coords_out = jnp.copy(coordinates)
    out = _sc_table_gather(species.astype(jnp.int32), conv)
    return out.astype(conv_tensor.dtype), coords_out


# speedup vs baseline: 5.8281x; 1.0234x over previous
import functools

import jax
import jax.numpy as jnp
from jax import lax
from jax.experimental import pallas as pl
from jax.experimental.pallas import tpu as pltpu
from jax.experimental.pallas import tpu_sc as plsc

_L = 16
_NC = 2
_NS = 16
_NW = _NC * _NS
_ROWS = 8192
_COLS = 512
_ROWS_PER_W = _ROWS // _NW
_CROWS = 16
_NCHUNKS = _ROWS_PER_W // _CROWS
_CELEMS = _CROWS * _COLS
_NBUF = 4


def _sc_table_gather(species, conv):
    mesh = plsc.VectorSubcoreMesh(core_axis_name="c", subcore_axis_name="s")

    @functools.partial(
        pl.kernel,
        mesh=mesh,
        out_type=jax.ShapeDtypeStruct((_ROWS, _COLS), jnp.int32),
        compiler_params=pltpu.CompilerParams(needs_layout_passes=False),
        scratch_types=(
            [pltpu.VMEM((128,), jnp.int32)]
            + [pltpu.VMEM((_CROWS, _COLS), jnp.int32)] * (2 * _NBUF)
            + [pltpu.SemaphoreType.DMA] * (2 * _NBUF)
        ),
    )
    def k(species_hbm, conv_hbm, out_hbm, conv_v, *rest):
        in_bufs = rest[:_NBUF]
        out_bufs = rest[_NBUF:2 * _NBUF]
        in_sems = rest[2 * _NBUF:3 * _NBUF]
        out_sems = rest[3 * _NBUF:]
        wid = lax.axis_index("s") * _NC + lax.axis_index("c")
        base = wid * _ROWS_PER_W

        def start_in(ci):
            return pltpu.async_copy(
                species_hbm.at[pl.ds(base + ci * _CROWS, _CROWS)],
                in_bufs[ci % _NBUF], in_sems[ci % _NBUF])

        def start_out(ci):
            return pltpu.async_copy(
                out_bufs[ci % _NBUF],
                out_hbm.at[pl.ds(base + ci * _CROWS, _CROWS)],
                out_sems[ci % _NBUF])

        def translate(src, dst):
            @plsc.parallel_loop(0, _CELEMS, _L, unroll=8)
            def body(i):
                r = i >> 9
                c = i & (_COLS - 1)
                idx = src[r, pl.ds(c, _L)]
                dst[r, pl.ds(c, _L)] = plsc.load_gather(conv_v, [idx])

        pending_in = {ci: start_in(ci) for ci in range(_NBUF - 1)}
        pending_out = {}
        pltpu.sync_copy(conv_hbm, conv_v.at[pl.ds(0, 120)])
        for ci in range(_NCHUNKS):
            b = ci % _NBUF
            if ci + _NBUF - 1 < _NCHUNKS:
                pending_in[ci + _NBUF - 1] = start_in(ci + _NBUF - 1)
            pending_in.pop(ci).wait()
            if ci - _NBUF in pending_out:
                pending_out.pop(ci - _NBUF).wait()
            translate(in_bufs[b], out_bufs[b])
            pending_out[ci] = start_out(ci)
        for ci in sorted(pending_out):
            pending_out.pop(ci).wait()

    return k(species, conv)


def kernel(species, coordinates, conv_tensor):
    out = _sc_table_gather(species.astype(jnp.int32), conv_tensor.astype(jnp.int32))
    return out.astype(conv_tensor.dtype), coordinates


# speedup vs baseline: 5.8327x; 1.0008x over previous
import functools

import jax
import jax.numpy as jnp
from jax import lax
from jax.experimental import pallas as pl
from jax.experimental.pallas import tpu as pltpu
from jax.experimental.pallas import tpu_sc as plsc

_L = 16
_NC = 2
_NS = 16
_NW = _NC * _NS
_ROWS = 8192
_COLS = 512
_ROWS_PER_W = _ROWS // _NW
_CROWS = 32
_NCHUNKS = _ROWS_PER_W // _CROWS
_CELEMS = _CROWS * _COLS
_NBUF = 3


def _sc_table_gather(species, conv):
    mesh = plsc.VectorSubcoreMesh(core_axis_name="c", subcore_axis_name="s")

    @functools.partial(
        pl.kernel,
        mesh=mesh,
        out_type=jax.ShapeDtypeStruct((_ROWS, _COLS), jnp.int32),
        compiler_params=pltpu.CompilerParams(needs_layout_passes=False),
        scratch_types=(
            [pltpu.VMEM((128,), jnp.int32)]
            + [pltpu.VMEM((_CROWS, _COLS), jnp.int32)] * (2 * _NBUF)
            + [pltpu.SemaphoreType.DMA] * (2 * _NBUF)
        ),
    )
    def k(species_hbm, conv_hbm, out_hbm, conv_v, *rest):
        in_bufs = rest[:_NBUF]
        out_bufs = rest[_NBUF:2 * _NBUF]
        in_sems = rest[2 * _NBUF:3 * _NBUF]
        out_sems = rest[3 * _NBUF:]
        wid = lax.axis_index("s") * _NC + lax.axis_index("c")
        base = wid * _ROWS_PER_W

        def start_in(ci):
            return pltpu.async_copy(
                species_hbm.at[pl.ds(base + ci * _CROWS, _CROWS)],
                in_bufs[ci % _NBUF], in_sems[ci % _NBUF])

        def start_out(ci):
            return pltpu.async_copy(
                out_bufs[ci % _NBUF],
                out_hbm.at[pl.ds(base + ci * _CROWS, _CROWS)],
                out_sems[ci % _NBUF])

        def translate(src, dst):
            @plsc.parallel_loop(0, _CELEMS, _L, unroll=8)
            def body(i):
                r = i >> 9
                c = i & (_COLS - 1)
                idx = src[r, pl.ds(c, _L)]
                dst[r, pl.ds(c, _L)] = plsc.load_gather(conv_v, [idx])

        pending_in = {ci: start_in(ci) for ci in range(_NBUF - 1)}
        pending_out = {}
        pltpu.sync_copy(conv_hbm, conv_v.at[pl.ds(0, 120)])
        for ci in range(_NCHUNKS):
            b = ci % _NBUF
            if ci + _NBUF - 1 < _NCHUNKS:
                pending_in[ci + _NBUF - 1] = start_in(ci + _NBUF - 1)
            pending_in.pop(ci).wait()
            if ci - _NBUF in pending_out:
                pending_out.pop(ci - _NBUF).wait()
            translate(in_bufs[b], out_bufs[b])
            pending_out[ci] = start_out(ci)
        for ci in sorted(pending_out):
            pending_out.pop(ci).wait()

    return k(species, conv)


def kernel(species, coordinates, conv_tensor):
    out = _sc_table_gather(species.astype(jnp.int32), conv_tensor.astype(jnp.int32))
    return out.astype(conv_tensor.dtype), coordinates
